# baseline (speedup 1.0000x reference)
import functools

import jax
import jax.numpy as jnp
from jax import lax
from jax.experimental import pallas as pl
from jax.experimental.pallas import tpu as pltpu
from jax.experimental.pallas import tpu_sc as plsc

_NC = 2
_NS = 16
_NW = _NC * _NS
_CH = 128


def _ceil_to(x, m):
    return (x + m - 1) // m * m


@functools.partial(jax.jit, static_argnames=("n_rows", "dim"))
def _gather_rows(table2d, idx_pad, n_rows, dim):
    span = n_rows // _NW
    rem = n_rows - span * _NW
    rem_pad = _ceil_to(rem, 8)
    n_chunks = -(-span // _CH)
    offs = [min(c * _CH, span - _CH) for c in range(n_chunks)]

    mesh = plsc.VectorSubcoreMesh(
        core_axis_name="c", subcore_axis_name="s",
        num_cores=_NC, num_subcores=_NS)

    scratch = [
        pltpu.VMEM((span,), jnp.int32),
        pltpu.VMEM((_CH, dim), table2d.dtype),
        pltpu.VMEM((_CH, dim), table2d.dtype),
        pltpu.SemaphoreType.DMA,
        pltpu.SemaphoreType.DMA,
    ]
    if rem:
        scratch += [
            pltpu.VMEM((rem_pad,), jnp.int32),
            pltpu.VMEM((rem_pad, dim), table2d.dtype),
        ]

    @functools.partial(
        pl.kernel,
        out_type=jax.ShapeDtypeStruct((n_rows, dim), table2d.dtype),
        mesh=mesh,
        scratch_types=scratch,
    )
    def run(table_hbm, idx_hbm, out_hbm, idx_v, rows0, rows1, sem0, sem1,
            *rem_scratch):
        wid = lax.axis_index("s") * _NC + lax.axis_index("c")
        base = wid * span
        rows = (rows0, rows1)
        sems = (sem0, sem1)

        pltpu.sync_copy(idx_hbm.at[pl.ds(base, span)], idx_v)

        def start_gather(c, b):
            pltpu.async_copy(
                table_hbm.at[idx_v.at[pl.ds(offs[c], _CH)]], rows[b], sems[b])

        start_gather(0, 0)
        for i in range(n_chunks):
            b = i & 1
            pltpu.make_async_copy(
                table_hbm.at[idx_v.at[pl.ds(offs[i], _CH)]], rows[b],
                sems[b]).wait()
            if i + 1 < n_chunks:
                start_gather(i + 1, b ^ 1)
            pltpu.sync_copy(rows[b], out_hbm.at[pl.ds(base + offs[i], _CH)])

        if rem:
            idx_t, rows_t = rem_scratch
            tbase = span * _NW

            @pl.when(wid == _NW - 1)
            def _():
                pltpu.sync_copy(idx_hbm.at[pl.ds(tbase, rem_pad)], idx_t)
                pltpu.async_copy(table_hbm.at[idx_t], rows_t, sem0).wait()
                pltpu.sync_copy(rows_t.at[pl.ds(0, rem)],
                                out_hbm.at[pl.ds(tbase, rem)])

    return run(table2d, idx_pad)


def _make_transpose_body(q2, g, dim):
    def _transpose_body(x_ref, o_ref):
        for i in range(g):
            o_ref[pl.ds(i * dim, dim), :] = x_ref[i, :q2, :].T
    return _transpose_body


_G = 16


def _make_transpose_chunk(q2, q2p, dim, s, ln, q1):
    def _transpose_body(x_ref, *refs):
        o_ref = refs[-1]
        for i in range(_G):
            o_ref[pl.ds(i * dim, dim), :] = x_ref[i, :q2, :].T

    nb = -(-ln // _G)
    sb = s // _G
    in_specs = [pl.BlockSpec((_G, q2p, dim), lambda a: (a, 0, 0))]
    io_alias = {}
    if s:
        in_specs.append(pl.BlockSpec(memory_space=pl.ANY))
        io_alias = {1: 0}
    return pl.pallas_call(
        _transpose_body,
        grid=(nb,),
        in_specs=in_specs,
        out_specs=pl.BlockSpec((_G * dim, q2), lambda a: (a + sb, 0)),
        out_shape=jax.ShapeDtypeStruct((q1 * dim, q2), jnp.float32),
        input_output_aliases=io_alias,
    )


@functools.partial(jax.jit, static_argnames=("q1", "q2", "dim", "nsplit"))
def _gather_transpose(table2d, idx2p, q1, q2, dim, nsplit):
    q2p = idx2p.shape[1]
    bounds = [0]
    per = _ceil_to(q1 // nsplit, 32)
    while bounds[-1] + 2 * per <= q1:
        bounds.append(bounds[-1] + per)
    bounds.append(q1)
    out2 = None
    for s, e in zip(bounds[:-1], bounds[1:]):
        ln = e - s
        g2d = _gather_rows(table2d, idx2p[s:e].reshape(-1),
                           n_rows=ln * q2p, dim=dim)
        g3d = g2d.reshape(ln, q2p, dim)
        call = _make_transpose_chunk(q2, q2p, dim, s, ln, q1)
        out2 = call(g3d) if s == 0 else call(g3d, out2)
    return out2


def kernel(unique_params, index_map):
    n, di, dj = unique_params.shape
    dim = di * dj
    q1, q2 = index_map.shape
    table2d = unique_params.reshape(n, dim)
    idx = index_map.astype(jnp.int32)
    q2p = _ceil_to(q2, 8)
    if q2p != q2:
        idx = jnp.pad(idx, ((0, 0), (0, q2p - q2)), mode="wrap")
    out2 = _gather_transpose(table2d, idx, q1=q1, q2=q2, dim=dim, nsplit=2)
    return out2.reshape(q1, di, dj, q2).transpose(0, 3, 1, 2)


# speedup vs baseline: 1.1024x; 1.1024x over previous
import functools

import jax
import jax.numpy as jnp
from jax import lax
from jax.experimental import pallas as pl
from jax.experimental.pallas import tpu as pltpu
from jax.experimental.pallas import tpu_sc as plsc

_NC = 2
_NS = 16
_NW = _NC * _NS
_CH = 128


def _ceil_to(x, m):
    return (x + m - 1) // m * m


@functools.partial(jax.jit, static_argnames=("n_rows", "dim"))
def _gather_rows(table2d, idx_pad, n_rows, dim):
    span = n_rows // _NW
    rem = n_rows - span * _NW
    rem_pad = _ceil_to(rem, 8)
    n_chunks = -(-span // _CH)
    offs = [min(c * _CH, span - _CH) for c in range(n_chunks)]

    mesh = plsc.VectorSubcoreMesh(
        core_axis_name="c", subcore_axis_name="s",
        num_cores=_NC, num_subcores=_NS)

    scratch = [
        pltpu.VMEM((span,), jnp.int32),
        pltpu.VMEM((_CH, dim), table2d.dtype),
        pltpu.VMEM((_CH, dim), table2d.dtype),
        pltpu.SemaphoreType.DMA,
        pltpu.SemaphoreType.DMA,
    ]
    if rem:
        scratch += [
            pltpu.VMEM((rem_pad,), jnp.int32),
            pltpu.VMEM((rem_pad, dim), table2d.dtype),
        ]

    @functools.partial(
        pl.kernel,
        out_type=jax.ShapeDtypeStruct((n_rows, dim), table2d.dtype),
        mesh=mesh,
        scratch_types=scratch,
    )
    def run(table_hbm, idx_hbm, out_hbm, idx_v, rows0, rows1, sem0, sem1,
            *rem_scratch):
        wid = lax.axis_index("s") * _NC + lax.axis_index("c")
        base = wid * span
        rows = (rows0, rows1)
        sems = (sem0, sem1)

        pltpu.sync_copy(idx_hbm.at[pl.ds(base, span)], idx_v)

        def start_gather(c, b):
            pltpu.async_copy(
                table_hbm.at[idx_v.at[pl.ds(offs[c], _CH)]], rows[b], sems[b])

        start_gather(0, 0)
        for i in range(n_chunks):
            b = i & 1
            pltpu.make_async_copy(
                table_hbm.at[idx_v.at[pl.ds(offs[i], _CH)]], rows[b],
                sems[b]).wait()
            if i + 1 < n_chunks:
                start_gather(i + 1, b ^ 1)
            pltpu.sync_copy(rows[b], out_hbm.at[pl.ds(base + offs[i], _CH)])

        if rem:
            idx_t, rows_t = rem_scratch
            tbase = span * _NW

            @pl.when(wid == _NW - 1)
            def _():
                pltpu.sync_copy(idx_hbm.at[pl.ds(tbase, rem_pad)], idx_t)
                pltpu.async_copy(table_hbm.at[idx_t], rows_t, sem0).wait()
                pltpu.sync_copy(rows_t.at[pl.ds(0, rem)],
                                out_hbm.at[pl.ds(tbase, rem)])

    return run(table2d, idx_pad)


def _make_transpose_body(q2, g, dim):
    def _transpose_body(x_ref, o_ref):
        for i in range(g):
            o_ref[pl.ds(i * dim, dim), :] = x_ref[i, :q2, :].T
    return _transpose_body


@functools.partial(jax.jit, static_argnames=("q1", "q2", "dim"))
def _transpose_blocks(g3d, q1, q2, dim):
    q2p = g3d.shape[1]
    g = 32
    assert q1 % g != -1
    ng = -(-q1 // g)
    return pl.pallas_call(
        _make_transpose_body(q2, g, dim),
        grid=(ng,),
        in_specs=[pl.BlockSpec((g, q2p, dim), lambda a: (a, 0, 0))],
        out_specs=pl.BlockSpec((g * dim, q2), lambda a: (a, 0)),
        out_shape=jax.ShapeDtypeStruct((q1 * dim, q2), g3d.dtype),
    )(g3d)


def kernel(unique_params, index_map):
    n, di, dj = unique_params.shape
    dim = di * dj
    q1, q2 = index_map.shape
    table2d = unique_params.reshape(n, dim)
    idx = index_map.astype(jnp.int32)
    q2p = _ceil_to(q2, 8)
    if q2p != q2:
        idx = jnp.pad(idx, ((0, 0), (0, q2p - q2)), mode="wrap")
    g2d = _gather_rows(table2d, idx.reshape(-1), n_rows=q1 * q2p, dim=dim)
    out2 = _transpose_blocks(g2d.reshape(q1, q2p, dim), q1=q1, q2=q2, dim=dim)
    return out2.reshape(q1, di, dj, q2).transpose(0, 3, 1, 2)
